# baseline (speedup 1.0000x reference)
import functools

import jax
import jax.numpy as jnp
from jax import lax
from jax.experimental import pallas as pl
from jax.experimental.pallas import tpu as pltpu
from jax.experimental.pallas import tpu_sc as plsc

NC, NS = 2, 16
NW = NC * NS
GC = 80
SB = 80
NORM = 100.0


def _pre_body(h_ref, w1a_ref, w1b_ref, t_ref):
    h = h_ref[...]
    t_ref[0, :, :] = jnp.dot(h, w1a_ref[...], preferred_element_type=jnp.float32)
    t_ref[1, :, :] = jnp.dot(h, w1b_ref[...], preferred_element_type=jnp.float32)


def _precompute(h, w1a, w1b):
    n, hid = h.shape
    bn = 2000
    return pl.pallas_call(
        _pre_body,
        grid=(n // bn,),
        in_specs=[
            pl.BlockSpec((bn, hid), lambda i: (i, 0)),
            pl.BlockSpec((hid, hid), lambda i: (0, 0)),
            pl.BlockSpec((hid, hid), lambda i: (0, 0)),
        ],
        out_specs=pl.BlockSpec((2, bn, hid), lambda i: (0, i, 0)),
        out_shape=jax.ShapeDtypeStruct((2, n, hid), jnp.float32),
    )(h, w1a, w1b)


def _gather(table, gidx):
    tot, hid = gidx.shape[0], table.shape[1]
    dt = table.dtype
    per_w = tot // NW
    nch = per_w // GC
    mesh = plsc.VectorSubcoreMesh(
        core_axis_name="c", subcore_axis_name="s", num_cores=NC, num_subcores=NS)

    npair = nch // 2

    @functools.partial(
        pl.kernel, mesh=mesh,
        out_type=jax.ShapeDtypeStruct((tot, hid), dt),
        scratch_types=[
            pltpu.VMEM((per_w,), jnp.int32),
            pltpu.VMEM((GC, hid), dt),
            pltpu.VMEM((GC, hid), dt),
            pltpu.SemaphoreType.DMA,
            pltpu.SemaphoreType.DMA,
            pltpu.SemaphoreType.DMA,
            pltpu.SemaphoreType.DMA,
        ],
        compiler_params=pltpu.CompilerParams(use_tc_tiling_on_sc=False),
    )
    def k(tbl_hbm, idx_hbm, out_hbm, idx_v, rows_a, rows_b, sga, sgb, soa, sob):
        wid = lax.axis_index("s") * NC + lax.axis_index("c")
        base = wid * per_w
        pltpu.sync_copy(idx_hbm.at[pl.ds(base, per_w)], idx_v)

        def pair(k_, carry):
            ja = 2 * k_
            jb = 2 * k_ + 1

            @pl.when(k_ > 0)
            def _():
                pltpu.make_async_copy(rows_a, out_hbm.at[pl.ds(base, GC)], soa).wait()
                pltpu.make_async_copy(rows_b, out_hbm.at[pl.ds(base, GC)], sob).wait()

            ga = pltpu.async_copy(
                tbl_hbm.at[idx_v.at[pl.ds(ja * GC, GC)]], rows_a, sga)
            gb = pltpu.async_copy(
                tbl_hbm.at[idx_v.at[pl.ds(jb * GC, GC)]], rows_b, sgb)
            ga.wait()
            pltpu.async_copy(rows_a, out_hbm.at[pl.ds(base + ja * GC, GC)], soa)
            gb.wait()
            pltpu.async_copy(rows_b, out_hbm.at[pl.ds(base + jb * GC, GC)], sob)
            return carry

        lax.fori_loop(0, npair, pair, 0)
        pltpu.make_async_copy(rows_a, out_hbm.at[pl.ds(base, GC)], soa).wait()
        pltpu.make_async_copy(rows_b, out_hbm.at[pl.ds(base, GC)], sob).wait()

    return k(table, gidx)


def _unpack_bf16_pairs(gi):
    lo = lax.bitcast_convert_type(lax.shift_left(gi, 16), jnp.float32)
    hi = lax.bitcast_convert_type(jnp.bitwise_and(gi, jnp.int32(-65536)),
                                  jnp.float32)
    return jnp.concatenate([lo, hi], axis=1)


def _mlp_body(g_ref, ea_ref, cd_ref, w1c_ref, b1_ref, w2_ref, b2_ref, w3_ref,
              t0_ref, t1_ref, t2_ref):
    g = (_unpack_bf16_pairs(g_ref[0, :, :]) + _unpack_bf16_pairs(g_ref[1, :, :])
         + b1_ref[...]
         + jnp.dot(ea_ref[...], w1c_ref[...], preferred_element_type=jnp.float32))
    x1 = g * jax.nn.sigmoid(g)
    x2 = jnp.dot(x1.astype(jnp.bfloat16), w2_ref[...],
                 preferred_element_type=jnp.float32) + b2_ref[...]
    x2 = x2 * jax.nn.sigmoid(x2)
    out_t = lax.dot_general(w3_ref[...], x2, (((1,), (1,)), ((), ())),
                            preferred_element_type=jnp.float32)
    cd = cd_ref[...]
    t0_ref[...] = cd[0:1, :] * out_t
    t1_ref[...] = cd[1:2, :] * out_t
    t2_ref[...] = cd[2:3, :] * out_t


def _mlp(g2, ea, cdt, w1c, b1, w2, b2, w3row):
    _, e, hw = g2.shape
    hid = 2 * hw
    be = 1280
    ed = ea.shape[1]
    row_sds = jax.ShapeDtypeStruct((1, e), jnp.float32)
    return pl.pallas_call(
        _mlp_body,
        grid=(e // be,),
        in_specs=[
            pl.BlockSpec((2, be, hw), lambda i: (0, i, 0)),
            pl.BlockSpec((be, ed), lambda i: (i, 0)),
            pl.BlockSpec((3, be), lambda i: (0, i)),
            pl.BlockSpec((ed, hid), lambda i: (0, 0)),
            pl.BlockSpec((1, hid), lambda i: (0, 0)),
            pl.BlockSpec((hid, hid), lambda i: (0, 0)),
            pl.BlockSpec((1, hid), lambda i: (0, 0)),
            pl.BlockSpec((1, hid), lambda i: (0, 0)),
        ],
        out_specs=[
            pl.BlockSpec((1, be), lambda i: (0, i)),
            pl.BlockSpec((1, be), lambda i: (0, i)),
            pl.BlockSpec((1, be), lambda i: (0, i)),
        ],
        out_shape=[row_sds, row_sds, row_sds],
    )(g2, ea, cdt, w1c, b1, w2, b2, w3row)


def _scatter(tr0, tr1, tr2, row_r, zeros_n, n):
    e = tr0.shape[0]
    ew = e // NW
    nch = ew // SB
    rpt = 1000
    ntc = n // rpt
    mesh = plsc.VectorSubcoreMesh(
        core_axis_name="c", subcore_axis_name="s", num_cores=NC, num_subcores=NS)
    part = jax.ShapeDtypeStruct((n,), jnp.float32)

    @functools.partial(
        pl.kernel, mesh=mesh,
        out_type=[part] * 6,
        scratch_types=[
            pltpu.VMEM((ew // SB, SB), jnp.int32),
            pltpu.VMEM((ew,), jnp.float32),
            pltpu.VMEM((ew,), jnp.float32),
            pltpu.VMEM((ew,), jnp.float32),
            pltpu.VMEM_SHARED((n,), jnp.float32),
            pltpu.VMEM_SHARED((n,), jnp.float32),
            pltpu.VMEM_SHARED((n,), jnp.float32),
        ],
        compiler_params=pltpu.CompilerParams(use_tc_tiling_on_sc=False),
    )
    def k(tr0_hbm, tr1_hbm, tr2_hbm, rowr_hbm, z_hbm,
          o00, o01, o02, o10, o11, o12,
          idx_v, t0_v, t1_v, t2_v, a0, a1, a2):
        cid = lax.axis_index("c")
        sid = lax.axis_index("s")
        wid = sid * NC + cid

        @pl.when(sid == 0)
        def _():
            pltpu.sync_copy(z_hbm, a0)
            pltpu.sync_copy(z_hbm, a1)
            pltpu.sync_copy(z_hbm, a2)

        plsc.subcore_barrier()

        sl_in = pl.ds(wid * ew, ew)
        pltpu.sync_copy(rowr_hbm.at[wid], idx_v)
        pltpu.sync_copy(tr0_hbm.at[sl_in], t0_v)
        pltpu.sync_copy(tr1_hbm.at[sl_in], t1_v)
        pltpu.sync_copy(tr2_hbm.at[sl_in], t2_v)

        def chunk(j, carry):
            sl = pl.ds(j * SB, SB)
            pltpu.sync_copy(t0_v.at[sl], a0.at[idx_v.at[j]], add=True)
            pltpu.sync_copy(t1_v.at[sl], a1.at[idx_v.at[j]], add=True)
            pltpu.sync_copy(t2_v.at[sl], a2.at[idx_v.at[j]], add=True)
            return carry

        lax.fori_loop(0, nch, chunk, 0)

        plsc.subcore_barrier()

        @pl.when(sid < ntc)
        def _():
            sl = pl.ds(sid * rpt, rpt)

            @pl.when(cid == 0)
            def _():
                pltpu.sync_copy(a0.at[sl], o00.at[sl])
                pltpu.sync_copy(a1.at[sl], o01.at[sl])
                pltpu.sync_copy(a2.at[sl], o02.at[sl])

            @pl.when(cid == 1)
            def _():
                pltpu.sync_copy(a0.at[sl], o10.at[sl])
                pltpu.sync_copy(a1.at[sl], o11.at[sl])
                pltpu.sync_copy(a2.at[sl], o12.at[sl])

    return k(tr0, tr1, tr2, row_r, zeros_n)


def kernel(h, coord, edge_index, coord_diff, edge_attr, W1, b1, W2, b2, W3):
    n, hid = h.shape
    e = edge_index.shape[1]
    row = edge_index[0]
    col = edge_index[1]

    t = _precompute(h, W1[:hid], W1[hid:2 * hid])
    tpk = lax.bitcast_convert_type(
        t.astype(jnp.bfloat16).reshape(2 * n, hid // 2, 2), jnp.int32)
    gidx = jnp.concatenate([row, col + n])
    g = _gather(tpk, gidx)

    perm = jnp.concatenate([jnp.arange(0, hid, 2), jnp.arange(1, hid, 2)])
    cdt = jnp.transpose(coord_diff)
    tr0, tr1, tr2 = _mlp(g.reshape(2, e, hid // 2), edge_attr, cdt,
                         W1[2 * hid:][:, perm], b1[perm].reshape(1, hid),
                         W2[perm, :].astype(jnp.bfloat16),
                         b2.reshape(1, hid), W3.reshape(1, hid))

    row_r = row.reshape(NW, e // NW // SB, SB)
    zeros_n = jnp.zeros((n,), jnp.float32)
    parts = _scatter(tr0.reshape(e), tr1.reshape(e), tr2.reshape(e),
                     row_r, zeros_n, n)

    agg = jnp.stack([parts[0] + parts[3],
                     parts[1] + parts[4],
                     parts[2] + parts[5]], axis=1) / NORM
    return coord + agg


# speedup vs baseline: 1.0960x; 1.0960x over previous
import functools

import jax
import jax.numpy as jnp
from jax import lax
from jax.experimental import pallas as pl
from jax.experimental.pallas import tpu as pltpu
from jax.experimental.pallas import tpu_sc as plsc

NC, NS = 2, 16
NW = NC * NS
GC = 80
SB = 80
NORM = 100.0


def _rne_bf16_bits(x):
    r = lax.bitcast_convert_type(x, jnp.int32)
    lsb = jnp.bitwise_and(lax.shift_right_logical(r, 16), jnp.int32(1))
    return r + jnp.int32(0x7FFF) + lsb


def _pack_halves(a):
    hw = a.shape[1] // 2
    lo = lax.shift_right_logical(_rne_bf16_bits(a[:, :hw]), 16)
    hi = jnp.bitwise_and(_rne_bf16_bits(a[:, hw:]), jnp.int32(-65536))
    return jnp.bitwise_or(hi, lo)


def _pre_body(h_ref, w1a_ref, w1b_ref, t_ref):
    h = h_ref[...]
    t_ref[0, :, :] = _pack_halves(
        jnp.dot(h, w1a_ref[...], preferred_element_type=jnp.float32))
    t_ref[1, :, :] = _pack_halves(
        jnp.dot(h, w1b_ref[...], preferred_element_type=jnp.float32))


def _precompute(h, w1a, w1b):
    n, hid = h.shape
    bn = 2000
    return pl.pallas_call(
        _pre_body,
        grid=(n // bn,),
        in_specs=[
            pl.BlockSpec((bn, hid), lambda i: (i, 0)),
            pl.BlockSpec((hid, hid), lambda i: (0, 0)),
            pl.BlockSpec((hid, hid), lambda i: (0, 0)),
        ],
        out_specs=pl.BlockSpec((2, bn, hid // 2), lambda i: (0, i, 0)),
        out_shape=jax.ShapeDtypeStruct((2, n, hid // 2), jnp.int32),
    )(h, w1a, w1b)


def _gather(table, gidx):
    tot, hid = gidx.shape[0], table.shape[1]
    dt = table.dtype
    per_w = tot // NW
    nch = per_w // GC
    mesh = plsc.VectorSubcoreMesh(
        core_axis_name="c", subcore_axis_name="s", num_cores=NC, num_subcores=NS)

    npair = nch // 2

    @functools.partial(
        pl.kernel, mesh=mesh,
        out_type=jax.ShapeDtypeStruct((tot, hid), dt),
        scratch_types=[
            pltpu.VMEM((per_w,), jnp.int32),
            pltpu.VMEM((GC, hid), dt),
            pltpu.VMEM((GC, hid), dt),
            pltpu.SemaphoreType.DMA,
            pltpu.SemaphoreType.DMA,
            pltpu.SemaphoreType.DMA,
            pltpu.SemaphoreType.DMA,
        ],
        compiler_params=pltpu.CompilerParams(use_tc_tiling_on_sc=False),
    )
    def k(tbl_hbm, idx_hbm, out_hbm, idx_v, rows_a, rows_b, sga, sgb, soa, sob):
        wid = lax.axis_index("s") * NC + lax.axis_index("c")
        base = wid * per_w
        pltpu.sync_copy(idx_hbm.at[pl.ds(base, per_w)], idx_v)

        def pair(k_, carry):
            ja = 2 * k_
            jb = 2 * k_ + 1

            @pl.when(k_ > 0)
            def _():
                pltpu.make_async_copy(rows_a, out_hbm.at[pl.ds(base, GC)], soa).wait()
                pltpu.make_async_copy(rows_b, out_hbm.at[pl.ds(base, GC)], sob).wait()

            ga = pltpu.async_copy(
                tbl_hbm.at[idx_v.at[pl.ds(ja * GC, GC)]], rows_a, sga)
            gb = pltpu.async_copy(
                tbl_hbm.at[idx_v.at[pl.ds(jb * GC, GC)]], rows_b, sgb)
            ga.wait()
            pltpu.async_copy(rows_a, out_hbm.at[pl.ds(base + ja * GC, GC)], soa)
            gb.wait()
            pltpu.async_copy(rows_b, out_hbm.at[pl.ds(base + jb * GC, GC)], sob)
            return carry

        lax.fori_loop(0, npair, pair, 0)
        pltpu.make_async_copy(rows_a, out_hbm.at[pl.ds(base, GC)], soa).wait()
        pltpu.make_async_copy(rows_b, out_hbm.at[pl.ds(base, GC)], sob).wait()

    return k(table, gidx)


def _unpack_bf16_pairs(gi):
    lo = lax.bitcast_convert_type(lax.shift_left(gi, 16), jnp.float32)
    hi = lax.bitcast_convert_type(jnp.bitwise_and(gi, jnp.int32(-65536)),
                                  jnp.float32)
    return jnp.concatenate([lo, hi], axis=1)


def _mlp_body(g_ref, ea_ref, cd_ref, w1c_ref, b1_ref, w2_ref, b2_ref, w3_ref,
              t0_ref, t1_ref, t2_ref):
    g = (_unpack_bf16_pairs(g_ref[0, :, :]) + _unpack_bf16_pairs(g_ref[1, :, :])
         + b1_ref[...]
         + jnp.dot(ea_ref[...], w1c_ref[...], preferred_element_type=jnp.float32))
    x1 = g * jax.nn.sigmoid(g)
    x2 = jnp.dot(x1.astype(jnp.bfloat16), w2_ref[...],
                 preferred_element_type=jnp.float32) + b2_ref[...]
    x2 = x2 * jax.nn.sigmoid(x2)
    out_t = lax.dot_general(w3_ref[...], x2, (((1,), (1,)), ((), ())),
                            preferred_element_type=jnp.float32)
    cd = cd_ref[...]
    t0_ref[...] = cd[0:1, :] * out_t
    t1_ref[...] = cd[1:2, :] * out_t
    t2_ref[...] = cd[2:3, :] * out_t


def _mlp(g2, ea, cdt, w1c, b1, w2, b2, w3row):
    _, e, hw = g2.shape
    hid = 2 * hw
    be = 1280
    ed = ea.shape[1]
    row_sds = jax.ShapeDtypeStruct((1, e), jnp.float32)
    return pl.pallas_call(
        _mlp_body,
        grid=(e // be,),
        in_specs=[
            pl.BlockSpec((2, be, hw), lambda i: (0, i, 0)),
            pl.BlockSpec((be, ed), lambda i: (i, 0)),
            pl.BlockSpec((3, be), lambda i: (0, i)),
            pl.BlockSpec((ed, hid), lambda i: (0, 0)),
            pl.BlockSpec((1, hid), lambda i: (0, 0)),
            pl.BlockSpec((hid, hid), lambda i: (0, 0)),
            pl.BlockSpec((1, hid), lambda i: (0, 0)),
            pl.BlockSpec((1, hid), lambda i: (0, 0)),
        ],
        out_specs=[
            pl.BlockSpec((1, be), lambda i: (0, i)),
            pl.BlockSpec((1, be), lambda i: (0, i)),
            pl.BlockSpec((1, be), lambda i: (0, i)),
        ],
        out_shape=[row_sds, row_sds, row_sds],
    )(g2, ea, cdt, w1c, b1, w2, b2, w3row)


def _scatter(tr0, tr1, tr2, row_r, zeros_n, n):
    e = tr0.shape[0]
    ew = e // NW
    nch = ew // SB
    rpt = 1000
    ntc = n // rpt
    mesh = plsc.VectorSubcoreMesh(
        core_axis_name="c", subcore_axis_name="s", num_cores=NC, num_subcores=NS)
    part = jax.ShapeDtypeStruct((n,), jnp.float32)

    @functools.partial(
        pl.kernel, mesh=mesh,
        out_type=[part] * 6,
        scratch_types=[
            pltpu.VMEM((ew // SB, SB), jnp.int32),
            pltpu.VMEM((ew,), jnp.float32),
            pltpu.VMEM((ew,), jnp.float32),
            pltpu.VMEM((ew,), jnp.float32),
            pltpu.VMEM_SHARED((n,), jnp.float32),
            pltpu.VMEM_SHARED((n,), jnp.float32),
            pltpu.VMEM_SHARED((n,), jnp.float32),
        ],
        compiler_params=pltpu.CompilerParams(use_tc_tiling_on_sc=False),
    )
    def k(tr0_hbm, tr1_hbm, tr2_hbm, rowr_hbm, z_hbm,
          o00, o01, o02, o10, o11, o12,
          idx_v, t0_v, t1_v, t2_v, a0, a1, a2):
        cid = lax.axis_index("c")
        sid = lax.axis_index("s")
        wid = sid * NC + cid

        @pl.when(sid == 0)
        def _():
            pltpu.sync_copy(z_hbm, a0)
            pltpu.sync_copy(z_hbm, a1)
            pltpu.sync_copy(z_hbm, a2)

        plsc.subcore_barrier()

        sl_in = pl.ds(wid * ew, ew)
        pltpu.sync_copy(rowr_hbm.at[wid], idx_v)
        pltpu.sync_copy(tr0_hbm.at[sl_in], t0_v)
        pltpu.sync_copy(tr1_hbm.at[sl_in], t1_v)
        pltpu.sync_copy(tr2_hbm.at[sl_in], t2_v)

        def chunk(j, carry):
            sl = pl.ds(j * SB, SB)
            pltpu.sync_copy(t0_v.at[sl], a0.at[idx_v.at[j]], add=True)
            pltpu.sync_copy(t1_v.at[sl], a1.at[idx_v.at[j]], add=True)
            pltpu.sync_copy(t2_v.at[sl], a2.at[idx_v.at[j]], add=True)
            return carry

        lax.fori_loop(0, nch, chunk, 0)

        plsc.subcore_barrier()

        @pl.when(sid < ntc)
        def _():
            sl = pl.ds(sid * rpt, rpt)

            @pl.when(cid == 0)
            def _():
                pltpu.sync_copy(a0.at[sl], o00.at[sl])
                pltpu.sync_copy(a1.at[sl], o01.at[sl])
                pltpu.sync_copy(a2.at[sl], o02.at[sl])

            @pl.when(cid == 1)
            def _():
                pltpu.sync_copy(a0.at[sl], o10.at[sl])
                pltpu.sync_copy(a1.at[sl], o11.at[sl])
                pltpu.sync_copy(a2.at[sl], o12.at[sl])

    return k(tr0, tr1, tr2, row_r, zeros_n)


def kernel(h, coord, edge_index, coord_diff, edge_attr, W1, b1, W2, b2, W3):
    n, hid = h.shape
    e = edge_index.shape[1]
    row = edge_index[0]
    col = edge_index[1]

    tpk = _precompute(h, W1[:hid], W1[hid:2 * hid]).reshape(2 * n, hid // 2)
    gidx = jnp.concatenate([row, col + n])
    g = _gather(tpk, gidx)

    cdt = jnp.transpose(coord_diff)
    tr0, tr1, tr2 = _mlp(g.reshape(2, e, hid // 2), edge_attr, cdt,
                         W1[2 * hid:], b1.reshape(1, hid),
                         W2.astype(jnp.bfloat16),
                         b2.reshape(1, hid), W3.reshape(1, hid))

    row_r = row.reshape(NW, e // NW // SB, SB)
    zeros_n = jnp.zeros((n,), jnp.float32)
    parts = _scatter(tr0.reshape(e), tr1.reshape(e), tr2.reshape(e),
                     row_r, zeros_n, n)

    agg = jnp.stack([parts[0] + parts[3],
                     parts[1] + parts[4],
                     parts[2] + parts[5]], axis=1) / NORM
    return coord + agg


# speedup vs baseline: 1.6542x; 1.5093x over previous
import functools

import jax
import jax.numpy as jnp
from jax import lax
from jax.experimental import pallas as pl
from jax.experimental.pallas import tpu as pltpu
from jax.experimental.pallas import tpu_sc as plsc

NC, NS = 2, 16
NW = NC * NS
GC = 80
SB = 80
NORM = 100.0


def _rne_bf16_bits(x):
    r = lax.bitcast_convert_type(x, jnp.int32)
    lsb = jnp.bitwise_and(lax.shift_right_logical(r, 16), jnp.int32(1))
    return r + jnp.int32(0x7FFF) + lsb


def _pack_halves(a):
    hw = a.shape[1] // 2
    lo = lax.shift_right_logical(_rne_bf16_bits(a[:, :hw]), 16)
    hi = jnp.bitwise_and(_rne_bf16_bits(a[:, hw:]), jnp.int32(-65536))
    return jnp.bitwise_or(hi, lo)


def _pre_body(h_ref, w1a_ref, w1b_ref, ta_ref, tb_ref):
    h = h_ref[...]
    ta_ref[...] = _pack_halves(
        jnp.dot(h, w1a_ref[...], preferred_element_type=jnp.float32))
    tb_ref[...] = _pack_halves(
        jnp.dot(h, w1b_ref[...], preferred_element_type=jnp.float32))


def _precompute(h, w1a, w1b):
    n, hid = h.shape
    bn = 2000
    tbl = jax.ShapeDtypeStruct((n, hid // 2), jnp.int32)
    return pl.pallas_call(
        _pre_body,
        grid=(n // bn,),
        in_specs=[
            pl.BlockSpec((bn, hid), lambda i: (i, 0)),
            pl.BlockSpec((hid, hid), lambda i: (0, 0)),
            pl.BlockSpec((hid, hid), lambda i: (0, 0)),
        ],
        out_specs=[
            pl.BlockSpec((bn, hid // 2), lambda i: (i, 0)),
            pl.BlockSpec((bn, hid // 2), lambda i: (i, 0)),
        ],
        out_shape=[tbl, tbl],
    )(h, w1a, w1b)


def _gather(tbl_a, tbl_b, rowi, coli):
    n64 = tbl_a.shape[1]
    e = rowi.shape[0]
    ew = e // NW
    nch = ew // GC
    npair = nch // 2
    mesh = plsc.VectorSubcoreMesh(
        core_axis_name="c", subcore_axis_name="s", num_cores=NC, num_subcores=NS)
    buf = pltpu.VMEM((GC, n64), jnp.int32)

    @functools.partial(
        pl.kernel, mesh=mesh,
        out_type=jax.ShapeDtypeStruct((e, 2 * n64), jnp.int32),
        scratch_types=[
            pltpu.VMEM((ew,), jnp.int32),
            pltpu.VMEM((ew,), jnp.int32),
            buf, buf, buf, buf,
        ] + [pltpu.SemaphoreType.DMA] * 8,
        compiler_params=pltpu.CompilerParams(use_tc_tiling_on_sc=False),
    )
    def k(ta, tb, ri_hbm, ci_hbm, out_hbm, ridx, cidx, ra_a, ra_b, rb_a, rb_b,
          sga_a, sga_b, sgb_a, sgb_b, soa_a, soa_b, sob_a, sob_b):
        wid = lax.axis_index("s") * NC + lax.axis_index("c")
        base = wid * ew
        pltpu.sync_copy(ri_hbm.at[pl.ds(base, ew)], ridx)
        pltpu.sync_copy(ci_hbm.at[pl.ds(base, ew)], cidx)

        def dst_a(j):
            return out_hbm.at[pl.ds(base + j * GC, GC), pl.ds(0, n64)]

        def dst_b(j):
            return out_hbm.at[pl.ds(base + j * GC, GC), pl.ds(n64, n64)]

        def fire_gathers(j, buf_a, buf_b, sa, sb):
            sl = pl.ds(j * GC, GC)
            return (pltpu.async_copy(ta.at[ridx.at[sl]], buf_a, sa),
                    pltpu.async_copy(tb.at[cidx.at[sl]], buf_b, sb))

        def drain_writes(buf_a, buf_b, sa, sb):
            pltpu.make_async_copy(buf_a, dst_a(0), sa).wait()
            pltpu.make_async_copy(buf_b, dst_b(0), sb).wait()

        def pair(k_, carry):
            ja = 2 * k_
            jb = 2 * k_ + 1

            @pl.when(k_ > 0)
            def _():
                drain_writes(ra_a, ra_b, soa_a, soa_b)
                drain_writes(rb_a, rb_b, sob_a, sob_b)

            ga = fire_gathers(ja, ra_a, ra_b, sga_a, sga_b)
            gb = fire_gathers(jb, rb_a, rb_b, sgb_a, sgb_b)
            ga[0].wait()
            ga[1].wait()
            pltpu.async_copy(ra_a, dst_a(ja), soa_a)
            pltpu.async_copy(ra_b, dst_b(ja), soa_b)
            gb[0].wait()
            gb[1].wait()
            pltpu.async_copy(rb_a, dst_a(jb), sob_a)
            pltpu.async_copy(rb_b, dst_b(jb), sob_b)
            return carry

        lax.fori_loop(0, npair, pair, 0)

        if nch % 2:
            jt = nch - 1
            drain_writes(ra_a, ra_b, soa_a, soa_b)
            ga = fire_gathers(jt, ra_a, ra_b, sga_a, sga_b)
            ga[0].wait()
            ga[1].wait()
            pltpu.async_copy(ra_a, dst_a(jt), soa_a)
            pltpu.async_copy(ra_b, dst_b(jt), soa_b)
        drain_writes(rb_a, rb_b, sob_a, sob_b)
        drain_writes(ra_a, ra_b, soa_a, soa_b)

    return k(tbl_a, tbl_b, rowi, coli)


def _unpack_bf16_pairs(gi):
    lo = lax.bitcast_convert_type(lax.shift_left(gi, 16), jnp.float32)
    hi = lax.bitcast_convert_type(jnp.bitwise_and(gi, jnp.int32(-65536)),
                                  jnp.float32)
    return jnp.concatenate([lo, hi], axis=1)


def _mlp_body(g_ref, ea_ref, cd_ref, w1c_ref, b1_ref, w2_ref, b2_ref, w3_ref,
              t0_ref, t1_ref, t2_ref):
    n64 = g_ref.shape[1] // 2
    g = (_unpack_bf16_pairs(g_ref[:, :n64]) + _unpack_bf16_pairs(g_ref[:, n64:])
         + b1_ref[...]
         + jnp.dot(ea_ref[...], w1c_ref[...], preferred_element_type=jnp.float32))
    x1 = g * jax.nn.sigmoid(g)
    x2 = jnp.dot(x1.astype(jnp.bfloat16), w2_ref[...],
                 preferred_element_type=jnp.float32) + b2_ref[...]
    x2 = x2 * jax.nn.sigmoid(x2)
    out_t = lax.dot_general(w3_ref[...], x2, (((1,), (1,)), ((), ())),
                            preferred_element_type=jnp.float32)
    cd = cd_ref[...]
    t0_ref[...] = cd[0:1, :] * out_t
    t1_ref[...] = cd[1:2, :] * out_t
    t2_ref[...] = cd[2:3, :] * out_t


def _mlp(g2, ea, cdt, w1c, b1, w2, b2, w3row):
    e, hid = g2.shape
    be = 1280
    ed = ea.shape[1]
    row_sds = jax.ShapeDtypeStruct((1, e), jnp.float32)
    return pl.pallas_call(
        _mlp_body,
        grid=(e // be,),
        in_specs=[
            pl.BlockSpec((be, hid), lambda i: (i, 0)),
            pl.BlockSpec((be, ed), lambda i: (i, 0)),
            pl.BlockSpec((3, be), lambda i: (0, i)),
            pl.BlockSpec((ed, hid), lambda i: (0, 0)),
            pl.BlockSpec((1, hid), lambda i: (0, 0)),
            pl.BlockSpec((hid, hid), lambda i: (0, 0)),
            pl.BlockSpec((1, hid), lambda i: (0, 0)),
            pl.BlockSpec((1, hid), lambda i: (0, 0)),
        ],
        out_specs=[
            pl.BlockSpec((1, be), lambda i: (0, i)),
            pl.BlockSpec((1, be), lambda i: (0, i)),
            pl.BlockSpec((1, be), lambda i: (0, i)),
        ],
        out_shape=[row_sds, row_sds, row_sds],
    )(g2, ea, cdt, w1c, b1, w2, b2, w3row)


def _scatter(tr0, tr1, tr2, row_r, zeros_n, n):
    e = tr0.shape[0]
    ew = e // NW
    nch = ew // SB
    rpt = 1000
    ntc = n // rpt
    mesh = plsc.VectorSubcoreMesh(
        core_axis_name="c", subcore_axis_name="s", num_cores=NC, num_subcores=NS)
    part = jax.ShapeDtypeStruct((n,), jnp.float32)

    @functools.partial(
        pl.kernel, mesh=mesh,
        out_type=[part] * 6,
        scratch_types=[
            pltpu.VMEM((ew // SB, SB), jnp.int32),
            pltpu.VMEM((ew,), jnp.float32),
            pltpu.VMEM((ew,), jnp.float32),
            pltpu.VMEM((ew,), jnp.float32),
            pltpu.VMEM_SHARED((n,), jnp.float32),
            pltpu.VMEM_SHARED((n,), jnp.float32),
            pltpu.VMEM_SHARED((n,), jnp.float32),
        ],
        compiler_params=pltpu.CompilerParams(use_tc_tiling_on_sc=False),
    )
    def k(tr0_hbm, tr1_hbm, tr2_hbm, rowr_hbm, z_hbm,
          o00, o01, o02, o10, o11, o12,
          idx_v, t0_v, t1_v, t2_v, a0, a1, a2):
        cid = lax.axis_index("c")
        sid = lax.axis_index("s")
        wid = sid * NC + cid

        @pl.when(sid == 0)
        def _():
            pltpu.sync_copy(z_hbm, a0)
            pltpu.sync_copy(z_hbm, a1)
            pltpu.sync_copy(z_hbm, a2)

        plsc.subcore_barrier()

        sl_in = pl.ds(wid * ew, ew)
        pltpu.sync_copy(rowr_hbm.at[wid], idx_v)
        pltpu.sync_copy(tr0_hbm.at[sl_in], t0_v)
        pltpu.sync_copy(tr1_hbm.at[sl_in], t1_v)
        pltpu.sync_copy(tr2_hbm.at[sl_in], t2_v)

        def chunk(j, carry):
            sl = pl.ds(j * SB, SB)
            pltpu.sync_copy(t0_v.at[sl], a0.at[idx_v.at[j]], add=True)
            pltpu.sync_copy(t1_v.at[sl], a1.at[idx_v.at[j]], add=True)
            pltpu.sync_copy(t2_v.at[sl], a2.at[idx_v.at[j]], add=True)
            return carry

        lax.fori_loop(0, nch, chunk, 0)

        plsc.subcore_barrier()

        @pl.when(sid < ntc)
        def _():
            sl = pl.ds(sid * rpt, rpt)

            @pl.when(cid == 0)
            def _():
                pltpu.sync_copy(a0.at[sl], o00.at[sl])
                pltpu.sync_copy(a1.at[sl], o01.at[sl])
                pltpu.sync_copy(a2.at[sl], o02.at[sl])

            @pl.when(cid == 1)
            def _():
                pltpu.sync_copy(a0.at[sl], o10.at[sl])
                pltpu.sync_copy(a1.at[sl], o11.at[sl])
                pltpu.sync_copy(a2.at[sl], o12.at[sl])

    return k(tr0, tr1, tr2, row_r, zeros_n)


def kernel(h, coord, edge_index, coord_diff, edge_attr, W1, b1, W2, b2, W3):
    n, hid = h.shape
    e = edge_index.shape[1]
    row = edge_index[0]
    col = edge_index[1]

    apk, bpk = _precompute(h, W1[:hid], W1[hid:2 * hid])
    g = _gather(apk, bpk, row, col)

    cdt = jnp.transpose(coord_diff)
    tr0, tr1, tr2 = _mlp(g, edge_attr, cdt,
                         W1[2 * hid:], b1.reshape(1, hid),
                         W2.astype(jnp.bfloat16),
                         b2.reshape(1, hid), W3.reshape(1, hid))

    row_r = row.reshape(NW, e // NW // SB, SB)
    zeros_n = jnp.zeros((n,), jnp.float32)
    parts = _scatter(tr0.reshape(e), tr1.reshape(e), tr2.reshape(e),
                     row_r, zeros_n, n)

    agg = jnp.stack([parts[0] + parts[3],
                     parts[1] + parts[4],
                     parts[2] + parts[5]], axis=1) / NORM
    return coord + agg


# speedup vs baseline: 1.7365x; 1.0497x over previous
import functools

import jax
import jax.numpy as jnp
from jax import lax
from jax.experimental import pallas as pl
from jax.experimental.pallas import tpu as pltpu
from jax.experimental.pallas import tpu_sc as plsc

NC, NS = 2, 16
NW = NC * NS
GC = 40
SB = 80
NORM = 100.0


def _rne_bf16_bits(x):
    r = lax.bitcast_convert_type(x, jnp.int32)
    lsb = jnp.bitwise_and(lax.shift_right_logical(r, 16), jnp.int32(1))
    return r + jnp.int32(0x7FFF) + lsb


def _pack_halves(a):
    hw = a.shape[1] // 2
    lo = lax.shift_right_logical(_rne_bf16_bits(a[:, :hw]), 16)
    hi = jnp.bitwise_and(_rne_bf16_bits(a[:, hw:]), jnp.int32(-65536))
    return jnp.bitwise_or(hi, lo)


def _pre_body(h_ref, w1a_ref, w1b_ref, ta_ref, tb_ref):
    h = h_ref[...]
    ta_ref[...] = _pack_halves(
        jnp.dot(h, w1a_ref[...], preferred_element_type=jnp.float32))
    tb_ref[...] = _pack_halves(
        jnp.dot(h, w1b_ref[...], preferred_element_type=jnp.float32))


def _precompute(h, w1a, w1b):
    n, hid = h.shape
    bn = 2000
    tbl = jax.ShapeDtypeStruct((n, hid // 2), jnp.int32)
    return pl.pallas_call(
        _pre_body,
        grid=(n // bn,),
        in_specs=[
            pl.BlockSpec((bn, hid), lambda i: (i, 0)),
            pl.BlockSpec((hid, hid), lambda i: (0, 0)),
            pl.BlockSpec((hid, hid), lambda i: (0, 0)),
        ],
        out_specs=[
            pl.BlockSpec((bn, hid // 2), lambda i: (i, 0)),
            pl.BlockSpec((bn, hid // 2), lambda i: (i, 0)),
        ],
        out_shape=[tbl, tbl],
    )(h, w1a, w1b)


def _gather(tbl_a, tbl_b, rowi, coli):
    n64 = tbl_a.shape[1]
    e = rowi.shape[0]
    ew = e // NW
    nch = ew // GC
    npair = nch // 2
    mesh = plsc.VectorSubcoreMesh(
        core_axis_name="c", subcore_axis_name="s", num_cores=NC, num_subcores=NS)
    buf = pltpu.VMEM((GC, n64), jnp.int32)

    @functools.partial(
        pl.kernel, mesh=mesh,
        out_type=jax.ShapeDtypeStruct((e, 2 * n64), jnp.int32),
        scratch_types=[
            pltpu.VMEM((ew,), jnp.int32),
            pltpu.VMEM((ew,), jnp.int32),
            buf, buf, buf, buf,
        ] + [pltpu.SemaphoreType.DMA] * 8,
        compiler_params=pltpu.CompilerParams(use_tc_tiling_on_sc=False),
    )
    def k(ta, tb, ri_hbm, ci_hbm, out_hbm, ridx, cidx, ra_a, ra_b, rb_a, rb_b,
          sga_a, sga_b, sgb_a, sgb_b, soa_a, soa_b, sob_a, sob_b):
        wid = lax.axis_index("s") * NC + lax.axis_index("c")
        base = wid * ew
        pltpu.sync_copy(ri_hbm.at[pl.ds(base, ew)], ridx)
        pltpu.sync_copy(ci_hbm.at[pl.ds(base, ew)], cidx)

        def dst_a(j):
            return out_hbm.at[pl.ds(base + j * GC, GC), pl.ds(0, n64)]

        def dst_b(j):
            return out_hbm.at[pl.ds(base + j * GC, GC), pl.ds(n64, n64)]

        def fire_gathers(j, buf_a, buf_b, sa, sb):
            sl = pl.ds(j * GC, GC)
            return (pltpu.async_copy(ta.at[ridx.at[sl]], buf_a, sa),
                    pltpu.async_copy(tb.at[cidx.at[sl]], buf_b, sb))

        def drain_writes(buf_a, buf_b, sa, sb):
            pltpu.make_async_copy(buf_a, dst_a(0), sa).wait()
            pltpu.make_async_copy(buf_b, dst_b(0), sb).wait()

        def pair(k_, carry):
            ja = 2 * k_
            jb = 2 * k_ + 1

            @pl.when(k_ > 0)
            def _():
                drain_writes(ra_a, ra_b, soa_a, soa_b)
                drain_writes(rb_a, rb_b, sob_a, sob_b)

            ga = fire_gathers(ja, ra_a, ra_b, sga_a, sga_b)
            gb = fire_gathers(jb, rb_a, rb_b, sgb_a, sgb_b)
            ga[0].wait()
            ga[1].wait()
            pltpu.async_copy(ra_a, dst_a(ja), soa_a)
            pltpu.async_copy(ra_b, dst_b(ja), soa_b)
            gb[0].wait()
            gb[1].wait()
            pltpu.async_copy(rb_a, dst_a(jb), sob_a)
            pltpu.async_copy(rb_b, dst_b(jb), sob_b)
            return carry

        lax.fori_loop(0, npair, pair, 0)

        if nch % 2:
            jt = nch - 1
            drain_writes(ra_a, ra_b, soa_a, soa_b)
            ga = fire_gathers(jt, ra_a, ra_b, sga_a, sga_b)
            ga[0].wait()
            ga[1].wait()
            pltpu.async_copy(ra_a, dst_a(jt), soa_a)
            pltpu.async_copy(ra_b, dst_b(jt), soa_b)
        drain_writes(rb_a, rb_b, sob_a, sob_b)
        drain_writes(ra_a, ra_b, soa_a, soa_b)

    return k(tbl_a, tbl_b, rowi, coli)


def _unpack_bf16_pairs(gi):
    lo = lax.bitcast_convert_type(lax.shift_left(gi, 16), jnp.float32)
    hi = lax.bitcast_convert_type(jnp.bitwise_and(gi, jnp.int32(-65536)),
                                  jnp.float32)
    return jnp.concatenate([lo, hi], axis=1)


def _mlp_body(g_ref, ea_ref, cd_ref, w1c_ref, b1_ref, w2_ref, b2_ref, w3_ref,
              t0_ref, t1_ref, t2_ref):
    n64 = g_ref.shape[1] // 2
    g = (_unpack_bf16_pairs(g_ref[:, :n64]) + _unpack_bf16_pairs(g_ref[:, n64:])
         + b1_ref[...]
         + jnp.dot(ea_ref[...], w1c_ref[...], preferred_element_type=jnp.float32))
    x1 = g * jax.nn.sigmoid(g)
    x2 = jnp.dot(x1.astype(jnp.bfloat16), w2_ref[...],
                 preferred_element_type=jnp.float32) + b2_ref[...]
    x2 = x2 * jax.nn.sigmoid(x2)
    out_t = lax.dot_general(w3_ref[...], x2, (((1,), (1,)), ((), ())),
                            preferred_element_type=jnp.float32)
    cd = cd_ref[...]
    t0_ref[...] = cd[0:1, :] * out_t
    t1_ref[...] = cd[1:2, :] * out_t
    t2_ref[...] = cd[2:3, :] * out_t


def _mlp(g2, ea, cdt, w1c, b1, w2, b2, w3row):
    e, hid = g2.shape
    be = 1280
    ed = ea.shape[1]
    row_sds = jax.ShapeDtypeStruct((1, e), jnp.float32)
    return pl.pallas_call(
        _mlp_body,
        grid=(e // be,),
        in_specs=[
            pl.BlockSpec((be, hid), lambda i: (i, 0)),
            pl.BlockSpec((be, ed), lambda i: (i, 0)),
            pl.BlockSpec((3, be), lambda i: (0, i)),
            pl.BlockSpec((ed, hid), lambda i: (0, 0)),
            pl.BlockSpec((1, hid), lambda i: (0, 0)),
            pl.BlockSpec((hid, hid), lambda i: (0, 0)),
            pl.BlockSpec((1, hid), lambda i: (0, 0)),
            pl.BlockSpec((1, hid), lambda i: (0, 0)),
        ],
        out_specs=[
            pl.BlockSpec((1, be), lambda i: (0, i)),
            pl.BlockSpec((1, be), lambda i: (0, i)),
            pl.BlockSpec((1, be), lambda i: (0, i)),
        ],
        out_shape=[row_sds, row_sds, row_sds],
    )(g2, ea, cdt, w1c, b1, w2, b2, w3row)


def _scatter(trh1, trh2, row_r, zeros_n, n):
    e = 2 * trh1[0].shape[0]
    ew = e // NW
    hw_workers = NW // 2
    nch = ew // SB
    rpt = 1000
    ntc = n // rpt
    mesh = plsc.VectorSubcoreMesh(
        core_axis_name="c", subcore_axis_name="s", num_cores=NC, num_subcores=NS)
    part = jax.ShapeDtypeStruct((n,), jnp.float32)

    @functools.partial(
        pl.kernel, mesh=mesh,
        out_type=[part] * 6,
        scratch_types=[
            pltpu.VMEM((ew // SB, SB), jnp.int32),
            pltpu.VMEM((ew,), jnp.float32),
            pltpu.VMEM((ew,), jnp.float32),
            pltpu.VMEM((ew,), jnp.float32),
            pltpu.VMEM_SHARED((n,), jnp.float32),
            pltpu.VMEM_SHARED((n,), jnp.float32),
            pltpu.VMEM_SHARED((n,), jnp.float32),
        ],
        compiler_params=pltpu.CompilerParams(use_tc_tiling_on_sc=False),
    )
    def k(h10_hbm, h11_hbm, h12_hbm, h20_hbm, h21_hbm, h22_hbm,
          rowr_hbm, z_hbm,
          o00, o01, o02, o10, o11, o12,
          idx_v, t0_v, t1_v, t2_v, a0, a1, a2):
        cid = lax.axis_index("c")
        sid = lax.axis_index("s")
        wid = sid * NC + cid

        @pl.when(sid == 0)
        def _():
            pltpu.sync_copy(z_hbm, a0)
            pltpu.sync_copy(z_hbm, a1)
            pltpu.sync_copy(z_hbm, a2)

        plsc.subcore_barrier()

        pltpu.sync_copy(rowr_hbm.at[wid], idx_v)

        @pl.when(wid < hw_workers)
        def _():
            sl_in = pl.ds(wid * ew, ew)
            pltpu.sync_copy(h10_hbm.at[sl_in], t0_v)
            pltpu.sync_copy(h11_hbm.at[sl_in], t1_v)
            pltpu.sync_copy(h12_hbm.at[sl_in], t2_v)

        @pl.when(wid >= hw_workers)
        def _():
            sl_in = pl.ds((wid - hw_workers) * ew, ew)
            pltpu.sync_copy(h20_hbm.at[sl_in], t0_v)
            pltpu.sync_copy(h21_hbm.at[sl_in], t1_v)
            pltpu.sync_copy(h22_hbm.at[sl_in], t2_v)

        def chunk(j, carry):
            sl = pl.ds(j * SB, SB)
            pltpu.sync_copy(t0_v.at[sl], a0.at[idx_v.at[j]], add=True)
            pltpu.sync_copy(t1_v.at[sl], a1.at[idx_v.at[j]], add=True)
            pltpu.sync_copy(t2_v.at[sl], a2.at[idx_v.at[j]], add=True)
            return carry

        lax.fori_loop(0, nch, chunk, 0)

        plsc.subcore_barrier()

        @pl.when(sid < ntc)
        def _():
            sl = pl.ds(sid * rpt, rpt)

            @pl.when(cid == 0)
            def _():
                pltpu.sync_copy(a0.at[sl], o00.at[sl])
                pltpu.sync_copy(a1.at[sl], o01.at[sl])
                pltpu.sync_copy(a2.at[sl], o02.at[sl])

            @pl.when(cid == 1)
            def _():
                pltpu.sync_copy(a0.at[sl], o10.at[sl])
                pltpu.sync_copy(a1.at[sl], o11.at[sl])
                pltpu.sync_copy(a2.at[sl], o12.at[sl])

    return k(*trh1, *trh2, row_r, zeros_n)


def kernel(h, coord, edge_index, coord_diff, edge_attr, W1, b1, W2, b2, W3):
    n, hid = h.shape
    e = edge_index.shape[1]
    row = edge_index[0]
    col = edge_index[1]

    apk, bpk = _precompute(h, W1[:hid], W1[hid:2 * hid])

    e2 = e // 2
    cdt = jnp.transpose(coord_diff)
    w2bf = W2.astype(jnp.bfloat16)
    halves = []
    for lo in (0, e2):
        gh = _gather(apk, bpk, lax.slice(row, (lo,), (lo + e2,)),
                     lax.slice(col, (lo,), (lo + e2,)))
        trh = _mlp(gh, lax.slice(edge_attr, (lo, 0), (lo + e2, edge_attr.shape[1])),
                   lax.slice(cdt, (0, lo), (3, lo + e2)),
                   W1[2 * hid:], b1.reshape(1, hid), w2bf,
                   b2.reshape(1, hid), W3.reshape(1, hid))
        halves.append([t.reshape(e2) for t in trh])

    row_r = row.reshape(NW, e // NW // SB, SB)
    zeros_n = jnp.zeros((n,), jnp.float32)
    parts = _scatter(halves[0], halves[1], row_r, zeros_n, n)

    agg = jnp.stack([parts[0] + parts[3],
                     parts[1] + parts[4],
                     parts[2] + parts[5]], axis=1) / NORM
    return coord + agg
